# Initial kernel scaffold; baseline (speedup 1.0000x reference)
import jax
import jax.numpy as jnp
from jax.experimental import pallas as pl


def kernel(x, edge_index, W1, b1, W2, b2):
    raise NotImplementedError("write your pallas kernel here")


# speedup vs baseline: 15.2463x; 15.2463x over previous
import functools

import jax
import jax.numpy as jnp
from jax import lax
from jax.experimental import pallas as pl
from jax.experimental.pallas import tpu as pltpu
from jax.experimental.pallas import tpu_sc as plsc

N = 10000
E = 320000
D_IN = 128
D_HID = 128
D_OUT = 64

NC = 2
NS = 16
NW = NC * NS
B = 128
G = -(-E // (NW * B))
E_PAD = NW * G * B
N_PAD = 10240
RT = N_PAD // NS
DUMMY = N

_mesh = functools.partial(
    plsc.VectorSubcoreMesh, core_axis_name="c", subcore_axis_name="s"
)


def _make_deg_kernel():
    @functools.partial(
        pl.kernel,
        out_type=jax.ShapeDtypeStruct((NC, N_PAD, 16), jnp.float32),
        mesh=_mesh(),
        scratch_types=[
            pltpu.VMEM((G, B), jnp.int32),
            pltpu.VMEM((B, 16), jnp.float32),
            pltpu.VMEM_SHARED((N_PAD, 16), jnp.float32),
        ],
        compiler_params=pltpu.CompilerParams(use_tc_tiling_on_sc=False),
    )
    def deg_kernel(dstg_hbm, ones_hbm, zeros_hbm, out_hbm, dst_v, ones_v, acc_sh):
        c = lax.axis_index("c")
        s = lax.axis_index("s")
        wid = s * NC + c
        rows = pl.ds(s * RT, RT)
        pltpu.sync_copy(zeros_hbm, acc_sh.at[rows])
        pltpu.sync_copy(dstg_hbm.at[wid], dst_v)
        pltpu.sync_copy(ones_hbm, ones_v)
        plsc.subcore_barrier()

        def body(g, carry):
            pltpu.sync_copy(ones_v, acc_sh.at[dst_v.at[g]], add=True)
            return carry

        lax.fori_loop(0, G, body, 0)
        plsc.subcore_barrier()
        pltpu.sync_copy(acc_sh.at[rows], out_hbm.at[c].at[rows])

    return deg_kernel


def _make_scatter_kernel(d):
    @functools.partial(
        pl.kernel,
        out_type=jax.ShapeDtypeStruct((NC, N_PAD, d), jnp.float32),
        mesh=_mesh(),
        scratch_types=[
            pltpu.VMEM((G, B), jnp.int32),
            pltpu.VMEM((G, B), jnp.int32),
            pltpu.VMEM((B, d), jnp.float32),
            pltpu.VMEM_SHARED((N_PAD, d), jnp.float32),
            pltpu.SemaphoreType.DMA,
        ],
        compiler_params=pltpu.CompilerParams(use_tc_tiling_on_sc=False),
    )
    def scatter_kernel(
        y_hbm, srcg_hbm, dstg_hbm, zeros_hbm, out_hbm, src_v, dst_v, buf_v, acc_sh, sem
    ):
        c = lax.axis_index("c")
        s = lax.axis_index("s")
        wid = s * NC + c
        rows = pl.ds(s * RT, RT)
        pltpu.sync_copy(zeros_hbm, acc_sh.at[rows])
        pltpu.sync_copy(srcg_hbm.at[wid], src_v)
        pltpu.sync_copy(dstg_hbm.at[wid], dst_v)
        plsc.subcore_barrier()

        def body(g, carry):
            pltpu.async_copy(y_hbm.at[src_v.at[g]], buf_v, sem).wait()
            pltpu.sync_copy(buf_v, acc_sh.at[dst_v.at[g]], add=True)
            return carry

        lax.fori_loop(0, G, body, 0)
        plsc.subcore_barrier()
        pltpu.sync_copy(acc_sh.at[rows], out_hbm.at[c].at[rows])

    return scatter_kernel


_deg = _make_deg_kernel()
_scatter_hid = _make_scatter_kernel(D_HID)
_scatter_out = _make_scatter_kernel(D_OUT)


R = 1000
GRID = N // R


def _dinv_block(degp):
    deg = degp[0, :, 0:1] + degp[1, :, 0:1] + 1.0
    return lax.rsqrt(deg)


def _t1_body(x_b, w_b, degp_b, y_b):
    dinv = _dinv_block(degp_b)
    h = jnp.dot(x_b[...], w_b[...], preferred_element_type=jnp.float32)
    y_b[...] = h * dinv


def _t2_body(zp_b, y1_b, b1_b, w2_b, degp_b, y2_b):
    dinv = _dinv_block(degp_b)
    pre = (zp_b[0] + zp_b[1] + y1_b[...]) * dinv + b1_b[...]
    h = jnp.maximum(pre, 0.0)
    y2_b[...] = jnp.dot(h, w2_b[...], preferred_element_type=jnp.float32) * dinv


def _t3_body(zp_b, y2_b, b2_b, degp_b, mu_b):
    dinv = _dinv_block(degp_b)
    mu_b[...] = (zp_b[0] + zp_b[1] + y2_b[...]) * dinv + b2_b[...]


def _row_spec(d):
    return pl.BlockSpec((R, d), lambda i: (i, 0))


def _part_spec(d):
    return pl.BlockSpec((NC, R, d), lambda i: (0, i, 0))


_full = lambda shape: pl.BlockSpec(shape, lambda i: tuple(0 for _ in shape))

_t1 = pl.pallas_call(
    _t1_body,
    grid=(GRID,),
    in_specs=[_row_spec(D_IN), _full((D_IN, D_HID)), _part_spec(16)],
    out_specs=_row_spec(D_HID),
    out_shape=jax.ShapeDtypeStruct((N, D_HID), jnp.float32),
)

_t2 = pl.pallas_call(
    _t2_body,
    grid=(GRID,),
    in_specs=[
        _part_spec(D_HID),
        _row_spec(D_HID),
        _full((1, D_HID)),
        _full((D_HID, D_OUT)),
        _part_spec(16),
    ],
    out_specs=_row_spec(D_OUT),
    out_shape=jax.ShapeDtypeStruct((N, D_OUT), jnp.float32),
)

_t3 = pl.pallas_call(
    _t3_body,
    grid=(GRID,),
    in_specs=[_part_spec(D_OUT), _row_spec(D_OUT), _full((1, D_OUT)), _part_spec(16)],
    out_specs=_row_spec(D_OUT),
    out_shape=jax.ShapeDtypeStruct((N, D_OUT), jnp.float32),
)


@jax.jit
def _run(x, edge_index, W1, b1, W2, b2):
    src = edge_index[0]
    dst = edge_index[1]
    pad = E_PAD - E
    srcg = jnp.concatenate([src, jnp.zeros((pad,), jnp.int32)]).reshape(NW, G, B)
    dstg = jnp.concatenate([dst, jnp.full((pad,), DUMMY, jnp.int32)]).reshape(NW, G, B)

    ones16 = jnp.ones((B, 16), jnp.float32)
    zeros16 = jnp.zeros((RT, 16), jnp.float32)
    zeros_hid = jnp.zeros((RT, D_HID), jnp.float32)
    zeros_out = jnp.zeros((RT, D_OUT), jnp.float32)

    degp = _deg(dstg, ones16, zeros16)
    y1 = _t1(x, W1, degp)
    z1p = _scatter_hid(y1, srcg, dstg, zeros_hid)
    y2 = _t2(z1p, y1, b1.reshape(1, D_HID), W2, degp)
    z2p = _scatter_out(y2, srcg, dstg, zeros_out)
    mu = _t3(z2p, y2, b2.reshape(1, D_OUT), degp)
    return mu


def kernel(x, edge_index, W1, b1, W2, b2):
    return _run(x, edge_index, W1, b1, W2, b2)
